# baseline (speedup 1.0000x reference)
import functools

import jax
import jax.numpy as jnp
from jax import lax
from jax.experimental import pallas as pl
from jax.experimental.pallas import tpu as pltpu
from jax.experimental.pallas import tpu_sc as plsc

N_NODES = 10000
D_FEAT = 128
D_HALF = 64
D_EDGE = 16
OUT_FEATURES = 128

NPAD = 10240
CHUNK = 128
KG = 160
KA = 80
EPAD = 16 * KG * CHUNK
RPT = NPAD // 16


def _sc_accumulate(xs, rowi, coli, attr):
    mesh = plsc.VectorSubcoreMesh(core_axis_name="c", subcore_axis_name="s")

    @functools.partial(
        pl.kernel,
        out_type=(
            jax.ShapeDtypeStruct((2, NPAD, D_HALF), jnp.float32),
            jax.ShapeDtypeStruct((2, NPAD, D_EDGE), jnp.float32),
            jax.ShapeDtypeStruct((2, NPAD, 16), jnp.float32),
        ),
        mesh=mesh,
        compiler_params=pltpu.CompilerParams(use_tc_tiling_on_sc=False),
        scratch_types=[
            pltpu.VMEM((4, CHUNK), jnp.int32),
            pltpu.VMEM((4, CHUNK), jnp.int32),
            pltpu.VMEM((4, CHUNK, D_HALF), jnp.float32),
            pltpu.VMEM((4, CHUNK, D_EDGE), jnp.float32),
            pltpu.VMEM((CHUNK, 16), jnp.float32),
            pltpu.VMEM_SHARED((NPAD, D_HALF), jnp.float32),
            pltpu.VMEM_SHARED((NPAD, D_EDGE), jnp.float32),
            pltpu.VMEM_SHARED((NPAD, 16), jnp.float32),
            [pltpu.SemaphoreType.DMA] * 4,
            [pltpu.SemaphoreType.DMA] * 4,
            [pltpu.SemaphoreType.DMA] * 4,
            [pltpu.SemaphoreType.DMA] * 4,
            [pltpu.SemaphoreType.DMA] * 4,
        ],
    )
    def kern(xs_hbm, rowi_hbm, coli_hbm, attr_hbm, g_out, a_out, cnt_out,
             ri, ci, xb, ab, ones_b, g_sp, a_sp, cnt_sp,
             semi, sema, semx, semg, semsa):
        c = lax.axis_index("c")
        s = lax.axis_index("s")

        @pl.loop(0, CHUNK)
        def _(r):
            ones_b[pl.ds(r, 1), pl.ds(0, 16)] = jnp.ones((1, 16), jnp.float32)
            ab[0, pl.ds(r, 1), pl.ds(0, 16)] = jnp.zeros((1, 16), jnp.float32)

            @pl.loop(0, D_HALF, step=16)
            def _(cc):
                xb[0, pl.ds(r, 1), pl.ds(cc, 16)] = jnp.zeros((1, 16), jnp.float32)

        for k in range(RPT // CHUNK):
            r0 = s * RPT + k * CHUNK
            pltpu.sync_copy(xb.at[0], g_sp.at[pl.ds(r0, CHUNK)])
            pltpu.sync_copy(ab.at[0], a_sp.at[pl.ds(r0, CHUNK)])
            pltpu.sync_copy(ab.at[0], cnt_sp.at[pl.ds(r0, CHUNK)])
        plsc.subcore_barrier()

        def fire_idx(j, k):
            pltpu.make_async_copy(rowi_hbm.at[s, j], ri.at[k], semi[k]).start()
            pltpu.make_async_copy(coli_hbm.at[s, j], ci.at[k], semi[k]).start()

            @pl.when(j // KA == c)
            def _():
                pltpu.make_async_copy(
                    attr_hbm.at[s, c, j - c * KA], ab.at[k], sema[k]).start()

        roff = c * N_NODES

        def wait_idx(j, k):
            pltpu.make_async_copy(rowi_hbm.at[s, j], ri.at[k], semi[k]).wait()
            pltpu.make_async_copy(coli_hbm.at[s, j], ci.at[k], semi[k]).wait()
            for o in range(0, CHUNK, 16):
                ri[k, pl.ds(o, 16)] = ri[k, pl.ds(o, 16)] + roff

        def fire_gather(j, k, b):
            pltpu.make_async_copy(xs_hbm.at[ri.at[k]], xb.at[b], semx[b]).start()

        def fire_scatter(j, k, b):
            pltpu.make_async_copy(xs_hbm.at[ri.at[k]], xb.at[b], semx[b]).wait()
            pltpu.async_copy(xb.at[b], g_sp.at[ci.at[k]], semg[b], add=True)

            @pl.when(j // KA == c)
            def _():
                pltpu.make_async_copy(
                    attr_hbm.at[s, c, j - c * KA], ab.at[k], sema[k]).wait()
                pltpu.async_copy(ab.at[k], a_sp.at[ci.at[k]], semsa[k], add=True)
                pltpu.async_copy(ones_b, cnt_sp.at[ci.at[k]], semsa[k], add=True)

        def wait_scatter(j, k, b):
            pltpu.make_async_copy(xb.at[b], g_sp.at[ci.at[k]], semg[b]).wait()

            @pl.when(j // KA == c)
            def _():
                pltpu.make_async_copy(ab.at[k], a_sp.at[ci.at[k]], semsa[k]).wait()
                pltpu.make_async_copy(ones_b, cnt_sp.at[ci.at[k]], semsa[k]).wait()

        for k in range(4):
            fire_idx(k, k)
        wait_idx(0, 0)
        fire_gather(0, 0, 0)
        wait_idx(1, 1)
        fire_gather(1, 1, 1)

        @pl.loop(0, KG, step=4)
        def _(j):
            fire_scatter(j, 0, 0)
            wait_idx(j + 2, 2)
            fire_gather(j + 2, 2, 2)
            fire_scatter(j + 1, 1, 1)
            wait_idx(j + 3, 3)
            fire_gather(j + 3, 3, 3)

            wait_scatter(j, 0, 0)

            @pl.when(j + 4 < KG)
            def _():
                fire_idx(j + 4, 0)

            fire_scatter(j + 2, 2, 2)
            wait_scatter(j + 1, 1, 1)

            @pl.when(j + 5 < KG)
            def _():
                fire_idx(j + 5, 1)

            fire_scatter(j + 3, 3, 3)
            wait_scatter(j + 2, 2, 2)

            @pl.when(j + 6 < KG)
            def _():
                fire_idx(j + 6, 2)

            @pl.when(j + 4 < KG)
            def _():
                wait_idx(j + 4, 0)
                fire_gather(j + 4, 0, 0)

            wait_scatter(j + 3, 3, 3)

            @pl.when(j + 7 < KG)
            def _():
                fire_idx(j + 7, 3)

            @pl.when(j + 5 < KG)
            def _():
                wait_idx(j + 5, 1)
                fire_gather(j + 5, 1, 1)

        plsc.subcore_barrier()

        r0 = s * RPT
        pltpu.sync_copy(g_sp.at[pl.ds(r0, RPT)], g_out.at[c, pl.ds(r0, RPT)])
        pltpu.sync_copy(a_sp.at[pl.ds(r0, RPT)], a_out.at[c, pl.ds(r0, RPT)])
        pltpu.sync_copy(cnt_sp.at[pl.ds(r0, RPT)], cnt_out.at[c, pl.ds(r0, RPT)])

    return kern(xs, rowi, coli, attr)


def _tc_finish(g, a, cnt, w1at, w1bt, w2t, bb):
    R = 1024
    grid = NPAD // R

    def body(g_ref, a_ref, c_ref, w1a_ref, w1b_ref, w2_ref, b_ref, o_ref):
        am = a_ref[0] + a_ref[1]
        cm = c_ref[0] + c_ref[1]
        cnt1 = cm[:, :1]
        y = jnp.dot(g_ref[0], w1a_ref[...], preferred_element_type=jnp.float32)
        y = y + jnp.dot(g_ref[1], w1b_ref[...], preferred_element_type=jnp.float32)
        y = y + jnp.dot(am, w2_ref[...], preferred_element_type=jnp.float32)
        y = y + cnt1 * b_ref[...]
        o_ref[...] = y / jnp.maximum(cnt1, 1.0)

    return pl.pallas_call(
        body,
        grid=(grid,),
        in_specs=[
            pl.BlockSpec((2, R, D_HALF), lambda i: (0, i, 0)),
            pl.BlockSpec((2, R, D_EDGE), lambda i: (0, i, 0)),
            pl.BlockSpec((2, R, 16), lambda i: (0, i, 0)),
            pl.BlockSpec((D_HALF, OUT_FEATURES), lambda i: (0, 0)),
            pl.BlockSpec((D_HALF, OUT_FEATURES), lambda i: (0, 0)),
            pl.BlockSpec((D_EDGE, OUT_FEATURES), lambda i: (0, 0)),
            pl.BlockSpec((1, OUT_FEATURES), lambda i: (0, 0)),
        ],
        out_specs=pl.BlockSpec((R, OUT_FEATURES), lambda i: (i, 0)),
        out_shape=jax.ShapeDtypeStruct((NPAD, OUT_FEATURES), jnp.float32),
    )(g, a, cnt, w1at, w1bt, w2t, bb)


def kernel(x, edge_index, edge_attr, W, b):
    row = edge_index[0].astype(jnp.int32)
    col = edge_index[1].astype(jnp.int32)
    e = row.shape[0]
    pad = EPAD - e
    row_p = jnp.concatenate([row, jnp.zeros((pad,), jnp.int32)])
    col_p = jnp.concatenate([col, jnp.full((pad,), NPAD - 1, jnp.int32)])
    attr_p = jnp.concatenate(
        [edge_attr, jnp.zeros((pad, D_EDGE), edge_attr.dtype)])

    xs = jnp.concatenate([x[:, :D_HALF], x[:, D_HALF:]], axis=0)
    rowi = row_p.reshape(16, KG, CHUNK)
    coli = col_p.reshape(16, KG, CHUNK)
    attr = attr_p.reshape(16, 2, KA, CHUNK, D_EDGE)

    g, a, cnt = _sc_accumulate(xs, rowi, coli, attr)

    w1at = W[:, :D_HALF].T
    w1bt = W[:, D_HALF:D_FEAT].T
    w2t = W[:, D_FEAT:].T
    bb = b.reshape(1, OUT_FEATURES)
    out_full = _tc_finish(g, a, cnt, w1at, w1bt, w2t, bb)
    return out_full[:N_NODES]


# speedup vs baseline: 1.1594x; 1.1594x over previous
import functools

import jax
import jax.numpy as jnp
from jax import lax
from jax.experimental import pallas as pl
from jax.experimental.pallas import tpu as pltpu
from jax.experimental.pallas import tpu_sc as plsc

N_NODES = 10000
D_FEAT = 128
D_HALF = 64
D_EDGE = 16
OUT_FEATURES = 128

NPAD = 10240
CHUNK = 128
KG = 160
KA = 80
EPAD = 16 * KG * CHUNK
RPT = NPAD // 16


def _sc_accumulate(xs, rc, attr):
    mesh = plsc.VectorSubcoreMesh(core_axis_name="c", subcore_axis_name="s")

    @functools.partial(
        pl.kernel,
        out_type=(
            jax.ShapeDtypeStruct((2, NPAD, D_HALF), jnp.float32),
            jax.ShapeDtypeStruct((2, NPAD, 32), jnp.float32),
        ),
        mesh=mesh,
        compiler_params=pltpu.CompilerParams(use_tc_tiling_on_sc=False),
        scratch_types=[
            pltpu.VMEM((4, 2, CHUNK), jnp.int32),
            pltpu.VMEM((4, CHUNK, D_HALF), jnp.float32),
            pltpu.VMEM((4, 16, 128), jnp.float32),
            pltpu.VMEM((4, CHUNK, 32), jnp.float32),
            pltpu.VMEM_SHARED((NPAD, D_HALF), jnp.float32),
            pltpu.VMEM_SHARED((NPAD, 32), jnp.float32),
            [pltpu.SemaphoreType.DMA] * 4,
            [pltpu.SemaphoreType.DMA] * 4,
            [pltpu.SemaphoreType.DMA] * 4,
            [pltpu.SemaphoreType.DMA] * 4,
            [pltpu.SemaphoreType.DMA] * 4,
        ],
    )
    def kern(xs_hbm, rc_hbm, attr_hbm, g_out, acnt_out,
             rc, xb, ab16, ab32, g_sp, acnt_sp,
             semi, sema, semx, semg, semsa):
        c = lax.axis_index("c")
        s = lax.axis_index("s")

        @pl.loop(0, CHUNK)
        def _(r):
            ab32[0, pl.ds(r, 1), pl.ds(0, 16)] = jnp.zeros((1, 16), jnp.float32)
            ab32[0, pl.ds(r, 1), pl.ds(16, 16)] = jnp.zeros((1, 16), jnp.float32)

            @pl.loop(0, D_HALF, step=16)
            def _(cc):
                xb[0, pl.ds(r, 1), pl.ds(cc, 16)] = jnp.zeros((1, 16), jnp.float32)

        for k in range(RPT // CHUNK):
            r0 = s * RPT + k * CHUNK
            pltpu.sync_copy(xb.at[0], g_sp.at[pl.ds(r0, CHUNK)])
            pltpu.sync_copy(ab32.at[0], acnt_sp.at[pl.ds(r0, CHUNK)])
        plsc.subcore_barrier()

        one0 = jnp.where(lax.iota(jnp.int32, 16) == 0,
                         jnp.float32(1.0), jnp.float32(0.0)).reshape(1, 16)
        for k in range(4):
            @pl.loop(0, CHUNK)
            def _(r, _k=k):
                ab32[_k, pl.ds(r, 1), pl.ds(16, 16)] = one0

        roff = c * N_NODES

        def fire_idx(j, k):
            pltpu.make_async_copy(rc_hbm.at[s, j], rc.at[k], semi[k]).start()

            @pl.when(j // KA == c)
            def _():
                pltpu.make_async_copy(
                    attr_hbm.at[s, c, j - c * KA], ab16.at[k], sema[k]).start()

        def wait_idx(j, k):
            pltpu.make_async_copy(rc_hbm.at[s, j], rc.at[k], semi[k]).wait()
            for o in range(0, CHUNK, 16):
                rc[k, 0, pl.ds(o, 16)] = rc[k, 0, pl.ds(o, 16)] + roff

        def fire_gather(j, k, b):
            pltpu.make_async_copy(xs_hbm.at[rc.at[k, 0]], xb.at[b], semx[b]).start()

        def fire_scatter(j, k, b):
            pltpu.make_async_copy(xs_hbm.at[rc.at[k, 0]], xb.at[b], semx[b]).wait()
            pltpu.async_copy(xb.at[b], g_sp.at[rc.at[k, 1]], semg[b], add=True)

            @pl.when(j // KA == c)
            def _():
                pltpu.make_async_copy(
                    attr_hbm.at[s, c, j - c * KA], ab16.at[k], sema[k]).wait()

                @pl.loop(0, 16)
                def _(r):
                    for g in range(8):
                        ab32[k, pl.ds(r * 8 + g, 1), pl.ds(0, 16)] = (
                            ab16[k, pl.ds(r, 1), pl.ds(16 * g, 16)])

                pltpu.async_copy(ab32.at[k], acnt_sp.at[rc.at[k, 1]],
                                 semsa[k], add=True)

        def wait_scatter(j, k, b):
            pltpu.make_async_copy(xb.at[b], g_sp.at[rc.at[k, 1]], semg[b]).wait()

            @pl.when(j // KA == c)
            def _():
                pltpu.make_async_copy(
                    ab32.at[k], acnt_sp.at[rc.at[k, 1]], semsa[k]).wait()

        for k in range(4):
            fire_idx(k, k)
        wait_idx(0, 0)
        fire_gather(0, 0, 0)
        wait_idx(1, 1)
        fire_gather(1, 1, 1)

        @pl.loop(0, KG, step=4)
        def _(j):
            fire_scatter(j, 0, 0)
            wait_idx(j + 2, 2)
            fire_gather(j + 2, 2, 2)
            fire_scatter(j + 1, 1, 1)
            wait_idx(j + 3, 3)
            fire_gather(j + 3, 3, 3)

            wait_scatter(j, 0, 0)

            @pl.when(j + 4 < KG)
            def _():
                fire_idx(j + 4, 0)

            fire_scatter(j + 2, 2, 2)
            wait_scatter(j + 1, 1, 1)

            @pl.when(j + 5 < KG)
            def _():
                fire_idx(j + 5, 1)

            fire_scatter(j + 3, 3, 3)
            wait_scatter(j + 2, 2, 2)

            @pl.when(j + 6 < KG)
            def _():
                fire_idx(j + 6, 2)

            @pl.when(j + 4 < KG)
            def _():
                wait_idx(j + 4, 0)
                fire_gather(j + 4, 0, 0)

            wait_scatter(j + 3, 3, 3)

            @pl.when(j + 7 < KG)
            def _():
                fire_idx(j + 7, 3)

            @pl.when(j + 5 < KG)
            def _():
                wait_idx(j + 5, 1)
                fire_gather(j + 5, 1, 1)

        plsc.subcore_barrier()

        r0 = s * RPT
        pltpu.sync_copy(g_sp.at[pl.ds(r0, RPT)], g_out.at[c, pl.ds(r0, RPT)])
        pltpu.sync_copy(acnt_sp.at[pl.ds(r0, RPT)], acnt_out.at[c, pl.ds(r0, RPT)])

    return kern(xs, rc, attr)


def _tc_finish(g, acnt, w1at, w1bt, w2t, bb):
    R = 1024
    grid = NPAD // R

    def body(g_ref, ac_ref, w1a_ref, w1b_ref, w2_ref, b_ref, o_ref):
        ac = ac_ref[0] + ac_ref[1]
        am = ac[:, :D_EDGE]
        cnt1 = ac[:, D_EDGE:D_EDGE + 1]
        y = jnp.dot(g_ref[0], w1a_ref[...], preferred_element_type=jnp.float32)
        y = y + jnp.dot(g_ref[1], w1b_ref[...], preferred_element_type=jnp.float32)
        y = y + jnp.dot(am, w2_ref[...], preferred_element_type=jnp.float32)
        y = y + cnt1 * b_ref[...]
        o_ref[...] = y / jnp.maximum(cnt1, 1.0)

    return pl.pallas_call(
        body,
        grid=(grid,),
        in_specs=[
            pl.BlockSpec((2, R, D_HALF), lambda i: (0, i, 0)),
            pl.BlockSpec((2, R, 32), lambda i: (0, i, 0)),
            pl.BlockSpec((D_HALF, OUT_FEATURES), lambda i: (0, 0)),
            pl.BlockSpec((D_HALF, OUT_FEATURES), lambda i: (0, 0)),
            pl.BlockSpec((D_EDGE, OUT_FEATURES), lambda i: (0, 0)),
            pl.BlockSpec((1, OUT_FEATURES), lambda i: (0, 0)),
        ],
        out_specs=pl.BlockSpec((R, OUT_FEATURES), lambda i: (i, 0)),
        out_shape=jax.ShapeDtypeStruct((NPAD, OUT_FEATURES), jnp.float32),
    )(g, acnt, w1at, w1bt, w2t, bb)


def kernel(x, edge_index, edge_attr, W, b):
    row = edge_index[0].astype(jnp.int32)
    col = edge_index[1].astype(jnp.int32)
    e = row.shape[0]
    pad = EPAD - e
    row_p = jnp.concatenate([row, jnp.zeros((pad,), jnp.int32)])
    col_p = jnp.concatenate([col, jnp.full((pad,), NPAD - 1, jnp.int32)])

    xs = jnp.concatenate([x[:, :D_HALF], x[:, D_HALF:]], axis=0)
    rowi = row_p.reshape(16, KG, CHUNK)
    coli = col_p.reshape(16, KG, CHUNK)
    rc = jnp.stack([rowi, coli], axis=2)

    attr8 = edge_attr.reshape(e // 8, 16 * 8)
    attr8p = jnp.concatenate(
        [attr8, jnp.zeros((pad // 8, 16 * 8), edge_attr.dtype)])
    attr = attr8p.reshape(16, 2, KA, 16, 128)

    g, acnt = _sc_accumulate(xs, rc, attr)

    w1at = W[:, :D_HALF].T
    w1bt = W[:, D_HALF:D_FEAT].T
    w2t = W[:, D_FEAT:].T
    bb = b.reshape(1, OUT_FEATURES)
    out_full = _tc_finish(g, acnt, w1at, w1bt, w2t, bb)
    return out_full[:N_NODES]


# speedup vs baseline: 1.1809x; 1.0185x over previous
import functools

import jax
import jax.numpy as jnp
from jax import lax
from jax.experimental import pallas as pl
from jax.experimental.pallas import tpu as pltpu
from jax.experimental.pallas import tpu_sc as plsc

N_NODES = 10000
D_FEAT = 128
D_HALF = 64
D_EDGE = 16
OUT_FEATURES = 128

NPAD = 10240
CHUNK = 128
KG = 160
KA = 80
EPAD = 16 * KG * CHUNK
RPT = NPAD // 16


def _sc_accumulate(xs, rc, attr):
    mesh = plsc.VectorSubcoreMesh(core_axis_name="c", subcore_axis_name="s")

    @functools.partial(
        pl.kernel,
        out_type=(
            jax.ShapeDtypeStruct((2, NPAD, D_HALF), jnp.float32),
            jax.ShapeDtypeStruct((2, NPAD, 32), jnp.float32),
        ),
        mesh=mesh,
        compiler_params=pltpu.CompilerParams(use_tc_tiling_on_sc=False),
        scratch_types=[
            pltpu.VMEM((4, 2, CHUNK), jnp.int32),
            pltpu.VMEM((4, CHUNK, D_HALF), jnp.float32),
            pltpu.VMEM((4, 16, 128), jnp.float32),
            pltpu.VMEM((4, CHUNK, 32), jnp.float32),
            pltpu.VMEM_SHARED((NPAD, D_HALF), jnp.float32),
            pltpu.VMEM_SHARED((NPAD, 32), jnp.float32),
            [pltpu.SemaphoreType.DMA] * 4,
            [pltpu.SemaphoreType.DMA] * 4,
            [pltpu.SemaphoreType.DMA] * 4,
            [pltpu.SemaphoreType.DMA] * 4,
            [pltpu.SemaphoreType.DMA] * 4,
        ],
    )
    def kern(xs_hbm, rc_hbm, attr_hbm, g_out, acnt_out,
             rc, xb, ab16, ab32, g_sp, acnt_sp,
             semi, sema, semx, semg, semsa):
        c = lax.axis_index("c")
        s = lax.axis_index("s")

        @pl.loop(0, CHUNK)
        def _(r):
            ab32[0, pl.ds(r, 1), pl.ds(0, 16)] = jnp.zeros((1, 16), jnp.float32)
            ab32[0, pl.ds(r, 1), pl.ds(16, 16)] = jnp.zeros((1, 16), jnp.float32)

            @pl.loop(0, D_HALF, step=16)
            def _(cc):
                xb[0, pl.ds(r, 1), pl.ds(cc, 16)] = jnp.zeros((1, 16), jnp.float32)

        for k in range(RPT // CHUNK):
            r0 = s * RPT + k * CHUNK
            pltpu.sync_copy(xb.at[0], g_sp.at[pl.ds(r0, CHUNK)])
            pltpu.sync_copy(ab32.at[0], acnt_sp.at[pl.ds(r0, CHUNK)])
        plsc.subcore_barrier()

        one0 = jnp.where(lax.iota(jnp.int32, 16) == 0,
                         jnp.float32(1.0), jnp.float32(0.0)).reshape(1, 16)
        for k in range(4):
            @pl.loop(0, CHUNK)
            def _(r, _k=k):
                ab32[_k, pl.ds(r, 1), pl.ds(16, 16)] = one0

        roff = c * N_NODES

        nblk = attr_hbm.shape[0]

        def fire_idx(j, k):
            pltpu.make_async_copy(rc_hbm.at[s, j], rc.at[k], semi[k]).start()
            gc = s * KG + j

            @pl.when((j // KA == c) & (gc < nblk))
            def _():
                pltpu.make_async_copy(
                    attr_hbm.at[gc], ab16.at[k], sema[k]).start()

        def wait_idx(j, k):
            pltpu.make_async_copy(rc_hbm.at[s, j], rc.at[k], semi[k]).wait()
            for o in range(0, CHUNK, 16):
                rc[k, 0, pl.ds(o, 16)] = rc[k, 0, pl.ds(o, 16)] + roff

        def fire_gather(j, k, b):
            pltpu.make_async_copy(xs_hbm.at[rc.at[k, 0]], xb.at[b], semx[b]).start()

        def fire_scatter(j, k, b):
            pltpu.make_async_copy(xs_hbm.at[rc.at[k, 0]], xb.at[b], semx[b]).wait()
            pltpu.async_copy(xb.at[b], g_sp.at[rc.at[k, 1]], semg[b], add=True)

            gc = s * KG + j

            @pl.when((j // KA == c) & (gc < nblk))
            def _():
                pltpu.make_async_copy(
                    attr_hbm.at[gc], ab16.at[k], sema[k]).wait()

                @pl.loop(0, 16)
                def _(r):
                    for g in range(8):
                        ab32[k, pl.ds(r * 8 + g, 1), pl.ds(0, 16)] = (
                            ab16[k, pl.ds(r, 1), pl.ds(16 * g, 16)])

                pltpu.async_copy(ab32.at[k], acnt_sp.at[rc.at[k, 1]],
                                 semsa[k], add=True)

        def wait_scatter(j, k, b):
            pltpu.make_async_copy(xb.at[b], g_sp.at[rc.at[k, 1]], semg[b]).wait()

            gc = s * KG + j

            @pl.when((j // KA == c) & (gc < nblk))
            def _():
                pltpu.make_async_copy(
                    ab32.at[k], acnt_sp.at[rc.at[k, 1]], semsa[k]).wait()

        for k in range(4):
            fire_idx(k, k)
        wait_idx(0, 0)
        fire_gather(0, 0, 0)
        wait_idx(1, 1)
        fire_gather(1, 1, 1)

        @pl.loop(0, KG, step=4)
        def _(j):
            fire_scatter(j, 0, 0)
            wait_idx(j + 2, 2)
            fire_gather(j + 2, 2, 2)
            fire_scatter(j + 1, 1, 1)
            wait_idx(j + 3, 3)
            fire_gather(j + 3, 3, 3)

            wait_scatter(j, 0, 0)

            @pl.when(j + 4 < KG)
            def _():
                fire_idx(j + 4, 0)

            fire_scatter(j + 2, 2, 2)
            wait_scatter(j + 1, 1, 1)

            @pl.when(j + 5 < KG)
            def _():
                fire_idx(j + 5, 1)

            fire_scatter(j + 3, 3, 3)
            wait_scatter(j + 2, 2, 2)

            @pl.when(j + 6 < KG)
            def _():
                fire_idx(j + 6, 2)

            @pl.when(j + 4 < KG)
            def _():
                wait_idx(j + 4, 0)
                fire_gather(j + 4, 0, 0)

            wait_scatter(j + 3, 3, 3)

            @pl.when(j + 7 < KG)
            def _():
                fire_idx(j + 7, 3)

            @pl.when(j + 5 < KG)
            def _():
                wait_idx(j + 5, 1)
                fire_gather(j + 5, 1, 1)

        plsc.subcore_barrier()

        r0 = s * RPT
        pltpu.sync_copy(g_sp.at[pl.ds(r0, RPT)], g_out.at[c, pl.ds(r0, RPT)])
        pltpu.sync_copy(acnt_sp.at[pl.ds(r0, RPT)], acnt_out.at[c, pl.ds(r0, RPT)])

    return kern(xs, rc, attr)


def _tc_finish(g, acnt, w1at, w1bt, w2t, bb):
    R = 1024
    grid = NPAD // R

    def body(g_ref, ac_ref, w1a_ref, w1b_ref, w2_ref, b_ref, o_ref):
        ac = ac_ref[0] + ac_ref[1]
        am = ac[:, :D_EDGE]
        cnt1 = ac[:, D_EDGE:D_EDGE + 1]
        y = jnp.dot(g_ref[0], w1a_ref[...], preferred_element_type=jnp.float32)
        y = y + jnp.dot(g_ref[1], w1b_ref[...], preferred_element_type=jnp.float32)
        y = y + jnp.dot(am, w2_ref[...], preferred_element_type=jnp.float32)
        y = y + cnt1 * b_ref[...]
        o_ref[...] = y / jnp.maximum(cnt1, 1.0)

    return pl.pallas_call(
        body,
        grid=(grid,),
        in_specs=[
            pl.BlockSpec((2, R, D_HALF), lambda i: (0, i, 0)),
            pl.BlockSpec((2, R, 32), lambda i: (0, i, 0)),
            pl.BlockSpec((D_HALF, OUT_FEATURES), lambda i: (0, 0)),
            pl.BlockSpec((D_HALF, OUT_FEATURES), lambda i: (0, 0)),
            pl.BlockSpec((D_EDGE, OUT_FEATURES), lambda i: (0, 0)),
            pl.BlockSpec((1, OUT_FEATURES), lambda i: (0, 0)),
        ],
        out_specs=pl.BlockSpec((R, OUT_FEATURES), lambda i: (i, 0)),
        out_shape=jax.ShapeDtypeStruct((NPAD, OUT_FEATURES), jnp.float32),
    )(g, acnt, w1at, w1bt, w2t, bb)


def kernel(x, edge_index, edge_attr, W, b):
    row = edge_index[0].astype(jnp.int32)
    col = edge_index[1].astype(jnp.int32)
    e = row.shape[0]
    pad = EPAD - e
    row_p = jnp.concatenate([row, jnp.zeros((pad,), jnp.int32)])
    col_p = jnp.concatenate([col, jnp.full((pad,), NPAD - 1, jnp.int32)])

    xs = jnp.concatenate([x[:, :D_HALF], x[:, D_HALF:]], axis=0)
    rowi = row_p.reshape(16, KG, CHUNK)
    coli = col_p.reshape(16, KG, CHUNK)
    rc = jnp.stack([rowi, coli], axis=2)

    attr = edge_attr.reshape(e // CHUNK, 16, 128)

    g, acnt = _sc_accumulate(xs, rc, attr)

    w1at = W[:, :D_HALF].T
    w1bt = W[:, D_HALF:D_FEAT].T
    w2t = W[:, D_FEAT:].T
    bb = b.reshape(1, OUT_FEATURES)
    out_full = _tc_finish(g, acnt, w1at, w1bt, w2t, bb)
    return out_full[:N_NODES]


# speedup vs baseline: 1.1986x; 1.0150x over previous
import functools

import jax
import jax.numpy as jnp
from jax import lax
from jax.experimental import pallas as pl
from jax.experimental.pallas import tpu as pltpu
from jax.experimental.pallas import tpu_sc as plsc

N_NODES = 10000
D_FEAT = 128
D_HALF = 64
D_EDGE = 16
OUT_FEATURES = 128

NPAD = 10240
CHUNK = 128
KG = 160
KA = 80
EPAD = 16 * KG * CHUNK
RPT = NPAD // 16


def _sc_accumulate(xs, rc, attr):
    mesh = plsc.VectorSubcoreMesh(core_axis_name="c", subcore_axis_name="s")

    @functools.partial(
        pl.kernel,
        out_type=(
            jax.ShapeDtypeStruct((2, NPAD, D_HALF), jnp.float32),
            jax.ShapeDtypeStruct((2, NPAD, D_EDGE), jnp.float32),
            jax.ShapeDtypeStruct((2, NPAD, 16), jnp.float32),
        ),
        mesh=mesh,
        compiler_params=pltpu.CompilerParams(use_tc_tiling_on_sc=False),
        scratch_types=[
            pltpu.VMEM((4, 2, CHUNK), jnp.int32),
            pltpu.VMEM((4, CHUNK, D_HALF), jnp.float32),
            pltpu.VMEM((4, CHUNK, D_EDGE), jnp.float32),
            pltpu.VMEM((CHUNK, 16), jnp.float32),
            pltpu.VMEM_SHARED((NPAD, D_HALF), jnp.float32),
            pltpu.VMEM_SHARED((NPAD, D_EDGE), jnp.float32),
            pltpu.VMEM_SHARED((NPAD, 16), jnp.float32),
            [pltpu.SemaphoreType.DMA] * 4,
            [pltpu.SemaphoreType.DMA] * 4,
            [pltpu.SemaphoreType.DMA] * 4,
            [pltpu.SemaphoreType.DMA] * 4,
            [pltpu.SemaphoreType.DMA] * 4,
        ],
    )
    def kern(xs_hbm, rc_hbm, attr_hbm, g_out, a_out, cnt_out,
             rc, xb, ab, ones_b, g_sp, a_sp, cnt_sp,
             semi, sema, semx, semg, semsa):
        c = lax.axis_index("c")
        s = lax.axis_index("s")

        @pl.loop(0, CHUNK)
        def _(r):
            ones_b[pl.ds(r, 1), pl.ds(0, 16)] = jnp.zeros((1, 16), jnp.float32)

            @pl.loop(0, D_HALF, step=16)
            def _(cc):
                xb[0, pl.ds(r, 1), pl.ds(cc, 16)] = jnp.zeros((1, 16), jnp.float32)

        for k in range(RPT // CHUNK):
            r0 = s * RPT + k * CHUNK
            pltpu.sync_copy(xb.at[0], g_sp.at[pl.ds(r0, CHUNK)])
            pltpu.sync_copy(ones_b, a_sp.at[pl.ds(r0, CHUNK)])
            pltpu.sync_copy(ones_b, cnt_sp.at[pl.ds(r0, CHUNK)])
        plsc.subcore_barrier()

        @pl.loop(0, CHUNK)
        def _(r):
            ones_b[pl.ds(r, 1), pl.ds(0, 16)] = jnp.ones((1, 16), jnp.float32)

        roff = c * N_NODES

        nblk = attr_hbm.shape[0] // CHUNK

        def fire_idx(j, k):
            pltpu.make_async_copy(rc_hbm.at[s, j], rc.at[k], semi[k]).start()
            gc = s * KG + j

            @pl.when((j // KA == c) & (gc < nblk))
            def _():
                pltpu.make_async_copy(
                    attr_hbm.at[pl.ds(gc * CHUNK, CHUNK)], ab.at[k],
                    sema[k]).start()

        def wait_idx(j, k):
            pltpu.make_async_copy(rc_hbm.at[s, j], rc.at[k], semi[k]).wait()
            for o in range(0, CHUNK, 16):
                rc[k, 0, pl.ds(o, 16)] = rc[k, 0, pl.ds(o, 16)] + roff

        def fire_gather(j, k, b):
            pltpu.make_async_copy(xs_hbm.at[rc.at[k, 0]], xb.at[b], semx[b]).start()

        def fire_scatter(j, k, b):
            pltpu.make_async_copy(xs_hbm.at[rc.at[k, 0]], xb.at[b], semx[b]).wait()
            pltpu.async_copy(xb.at[b], g_sp.at[rc.at[k, 1]], semg[b], add=True)

            gc = s * KG + j

            @pl.when((j // KA == c) & (gc < nblk))
            def _():
                pltpu.make_async_copy(
                    attr_hbm.at[pl.ds(gc * CHUNK, CHUNK)], ab.at[k],
                    sema[k]).wait()
                pltpu.async_copy(ab.at[k], a_sp.at[rc.at[k, 1]],
                                 semsa[k], add=True)
                pltpu.async_copy(ones_b, cnt_sp.at[rc.at[k, 1]],
                                 semsa[k], add=True)

        def wait_scatter(j, k, b):
            pltpu.make_async_copy(xb.at[b], g_sp.at[rc.at[k, 1]], semg[b]).wait()

            gc = s * KG + j

            @pl.when((j // KA == c) & (gc < nblk))
            def _():
                pltpu.make_async_copy(
                    ab.at[k], a_sp.at[rc.at[k, 1]], semsa[k]).wait()
                pltpu.make_async_copy(
                    ones_b, cnt_sp.at[rc.at[k, 1]], semsa[k]).wait()

        for k in range(4):
            fire_idx(k, k)
        wait_idx(0, 0)
        fire_gather(0, 0, 0)
        wait_idx(1, 1)
        fire_gather(1, 1, 1)

        @pl.loop(0, KG, step=4)
        def _(j):
            fire_scatter(j, 0, 0)
            wait_idx(j + 2, 2)
            fire_gather(j + 2, 2, 2)
            fire_scatter(j + 1, 1, 1)
            wait_idx(j + 3, 3)
            fire_gather(j + 3, 3, 3)

            wait_scatter(j, 0, 0)

            @pl.when(j + 4 < KG)
            def _():
                fire_idx(j + 4, 0)

            fire_scatter(j + 2, 2, 2)
            wait_scatter(j + 1, 1, 1)

            @pl.when(j + 5 < KG)
            def _():
                fire_idx(j + 5, 1)

            fire_scatter(j + 3, 3, 3)
            wait_scatter(j + 2, 2, 2)

            @pl.when(j + 6 < KG)
            def _():
                fire_idx(j + 6, 2)

            @pl.when(j + 4 < KG)
            def _():
                wait_idx(j + 4, 0)
                fire_gather(j + 4, 0, 0)

            wait_scatter(j + 3, 3, 3)

            @pl.when(j + 7 < KG)
            def _():
                fire_idx(j + 7, 3)

            @pl.when(j + 5 < KG)
            def _():
                wait_idx(j + 5, 1)
                fire_gather(j + 5, 1, 1)

        plsc.subcore_barrier()

        r0 = s * RPT
        pltpu.sync_copy(g_sp.at[pl.ds(r0, RPT)], g_out.at[c, pl.ds(r0, RPT)])
        pltpu.sync_copy(a_sp.at[pl.ds(r0, RPT)], a_out.at[c, pl.ds(r0, RPT)])
        pltpu.sync_copy(cnt_sp.at[pl.ds(r0, RPT)], cnt_out.at[c, pl.ds(r0, RPT)])

    return kern(xs, rc, attr)


def _tc_finish(g, a, cnt, w1at, w1bt, w2t, bb):
    R = 1024
    grid = NPAD // R

    def body(g_ref, a_ref, c_ref, w1a_ref, w1b_ref, w2_ref, b_ref, o_ref):
        am = a_ref[0] + a_ref[1]
        cm = c_ref[0] + c_ref[1]
        cnt1 = cm[:, :1]
        y = jnp.dot(g_ref[0], w1a_ref[...], preferred_element_type=jnp.float32)
        y = y + jnp.dot(g_ref[1], w1b_ref[...], preferred_element_type=jnp.float32)
        y = y + jnp.dot(am, w2_ref[...], preferred_element_type=jnp.float32)
        y = y + cnt1 * b_ref[...]
        o_ref[...] = y / jnp.maximum(cnt1, 1.0)

    return pl.pallas_call(
        body,
        grid=(grid,),
        in_specs=[
            pl.BlockSpec((2, R, D_HALF), lambda i: (0, i, 0)),
            pl.BlockSpec((2, R, D_EDGE), lambda i: (0, i, 0)),
            pl.BlockSpec((2, R, 16), lambda i: (0, i, 0)),
            pl.BlockSpec((D_HALF, OUT_FEATURES), lambda i: (0, 0)),
            pl.BlockSpec((D_HALF, OUT_FEATURES), lambda i: (0, 0)),
            pl.BlockSpec((D_EDGE, OUT_FEATURES), lambda i: (0, 0)),
            pl.BlockSpec((1, OUT_FEATURES), lambda i: (0, 0)),
        ],
        out_specs=pl.BlockSpec((R, OUT_FEATURES), lambda i: (i, 0)),
        out_shape=jax.ShapeDtypeStruct((NPAD, OUT_FEATURES), jnp.float32),
    )(g, a, cnt, w1at, w1bt, w2t, bb)


def kernel(x, edge_index, edge_attr, W, b):
    row = edge_index[0].astype(jnp.int32)
    col = edge_index[1].astype(jnp.int32)
    e = row.shape[0]
    pad = EPAD - e
    row_p = jnp.concatenate([row, jnp.zeros((pad,), jnp.int32)])
    col_p = jnp.concatenate([col, jnp.full((pad,), NPAD - 1, jnp.int32)])

    xs = jnp.concatenate([x[:, :D_HALF], x[:, D_HALF:]], axis=0)
    rowi = row_p.reshape(16, KG, CHUNK)
    coli = col_p.reshape(16, KG, CHUNK)
    rc = jnp.stack([rowi, coli], axis=2)

    g, a, cnt = _sc_accumulate(xs, rc, edge_attr)

    w1at = W[:, :D_HALF].T
    w1bt = W[:, D_HALF:D_FEAT].T
    w2t = W[:, D_FEAT:].T
    bb = b.reshape(1, OUT_FEATURES)
    out_full = _tc_finish(g, a, cnt, w1at, w1bt, w2t, bb)
    return out_full[:N_NODES]
